# baseline (speedup 1.0000x reference)
import functools

import jax
import jax.numpy as jnp
from jax import lax
from jax.experimental import pallas as pl
from jax.experimental.pallas import tpu as pltpu
from jax.experimental.pallas import tpu_sc as plsc

_N = 100000
_C = 128
_NBINS = 20

_BR = 2000
_NB = _N // _BR

_NC = 2
_NS = 16
_NW = _NC * _NS
_CHUNK = 3136
_NPAD = _CHUNK * _NW
_VECS = _CHUNK // 16


def _make_scale_kernel():
    mesh = plsc.VectorSubcoreMesh(core_axis_name="c", subcore_axis_name="s")

    @functools.partial(
        pl.kernel,
        mesh=mesh,
        out_type=jax.ShapeDtypeStruct((_NPAD,), jnp.float32),
        scratch_types=[
            pltpu.VMEM((_CHUNK,), jnp.int32),
            pltpu.VMEM((_CHUNK,), jnp.float32),
            pltpu.SemaphoreType.DMA,
        ],
    )
    def scale_k(bins_hbm, temp_hbm, out_hbm, idx_v, t_v, sem):
        wid = lax.axis_index("s") * _NC + lax.axis_index("c")
        base = wid * _CHUNK
        pltpu.sync_copy(bins_hbm.at[pl.ds(base, _CHUNK)], idx_v)
        pltpu.async_copy(temp_hbm.at[idx_v], t_v, sem).wait()

        def body(i, carry):
            t = t_v[pl.ds(i * 16, 16)]
            t_v[pl.ds(i * 16, 16)] = 1.0 / (jnp.maximum(t, 0.0) + 1e-8)
            return carry

        lax.fori_loop(0, _VECS, body, 0)
        pltpu.sync_copy(t_v, out_hbm.at[pl.ds(base, _CHUNK)])

    return scale_k


_scale_kernel = _make_scale_kernel()


def _tc_scale_body(scale_ref, logits_ref, out_ref):
    out_ref[...] = logits_ref[...] * scale_ref[...]


def kernel(logits, temperature, bin_assignments):
    bins = bin_assignments.astype(jnp.int32)
    bins_p = jnp.pad(bins, (0, _NPAD - _N))
    temp_p = jnp.pad(temperature.astype(jnp.float32), (0, 32 - _NBINS))
    scale = _scale_kernel(bins_p, temp_p)
    scale2d = scale.reshape(_NPAD, 1)
    out = pl.pallas_call(
        _tc_scale_body,
        grid=(_NB,),
        in_specs=[
            pl.BlockSpec((_BR, 1), lambda i: (i, 0)),
            pl.BlockSpec((_BR, _C), lambda i: (i, 0)),
        ],
        out_specs=pl.BlockSpec((_BR, _C), lambda i: (i, 0)),
        out_shape=jax.ShapeDtypeStruct((_N, _C), jnp.float32),
    )(scale2d, logits)
    return out


# speedup vs baseline: 5.3680x; 5.3680x over previous
import functools

import jax
import jax.numpy as jnp
from jax import lax
from jax.experimental import pallas as pl
from jax.experimental.pallas import tpu as pltpu
from jax.experimental.pallas import tpu_sc as plsc

_N = 100000
_C = 128
_NBINS = 20

_BR = 2000
_NB = _N // _BR

_NC = 2
_NS = 16
_NW = _NC * _NS
_CHUNK = 3136
_NPAD = _CHUNK * _NW
_VECS = _CHUNK // 16
_UNROLL = 4


def _make_scale_kernel():
    mesh = plsc.VectorSubcoreMesh(core_axis_name="c", subcore_axis_name="s")

    @functools.partial(
        pl.kernel,
        mesh=mesh,
        out_type=jax.ShapeDtypeStruct((_NPAD,), jnp.float32),
        scratch_types=[
            pltpu.VMEM((_CHUNK,), jnp.int32),
            pltpu.VMEM((_CHUNK,), jnp.float32),
            pltpu.VMEM((32,), jnp.float32),
        ],
    )
    def scale_k(bins_hbm, temp_hbm, out_hbm, idx_v, t_v, temp_v):
        wid = lax.axis_index("s") * _NC + lax.axis_index("c")
        base = wid * _CHUNK
        pltpu.sync_copy(temp_hbm, temp_v)
        pltpu.sync_copy(bins_hbm.at[pl.ds(base, _CHUNK)], idx_v)
        t0 = temp_v[pl.ds(0, 16)]
        t1 = temp_v[pl.ds(16, 16)]
        inv0 = 1.0 / (jnp.maximum(t0, 0.0) + 1e-8)
        inv1 = 1.0 / (jnp.maximum(t1, 0.0) + 1e-8)

        def body(i, carry):
            for u in range(_UNROLL):
                off = (i * _UNROLL + u) * 16
                idx = idx_v[pl.ds(off, 16)]
                g0 = inv0.at[jnp.minimum(idx, 15)].get(mode="promise_in_bounds")
                g1 = inv1.at[jnp.maximum(idx - 16, 0)].get(mode="promise_in_bounds")
                t_v[pl.ds(off, 16)] = jnp.where(idx < 16, g0, g1)
            return carry

        lax.fori_loop(0, _VECS // _UNROLL, body, 0)
        pltpu.sync_copy(t_v, out_hbm.at[pl.ds(base, _CHUNK)])

    return scale_k


_scale_kernel = _make_scale_kernel()


def _tc_scale_body(scale_ref, logits_ref, out_ref):
    out_ref[...] = logits_ref[...] * scale_ref[...]


def kernel(logits, temperature, bin_assignments):
    bins = bin_assignments.astype(jnp.int32)
    bins_p = jnp.pad(bins, (0, _NPAD - _N))
    temp_p = jnp.pad(temperature.astype(jnp.float32), (0, 32 - _NBINS))
    scale = _scale_kernel(bins_p, temp_p)
    scale2d = scale.reshape(_NPAD, 1)
    out = pl.pallas_call(
        _tc_scale_body,
        grid=(_NB,),
        in_specs=[
            pl.BlockSpec((_BR, 1), lambda i: (i, 0)),
            pl.BlockSpec((_BR, _C), lambda i: (i, 0)),
        ],
        out_specs=pl.BlockSpec((_BR, _C), lambda i: (i, 0)),
        out_shape=jax.ShapeDtypeStruct((_N, _C), jnp.float32),
    )(scale2d, logits)
    return out


# speedup vs baseline: 7.2497x; 1.3505x over previous
import functools

import jax
import jax.numpy as jnp
from jax import lax
from jax.experimental import pallas as pl
from jax.experimental.pallas import tpu as pltpu
from jax.experimental.pallas import tpu_sc as plsc

_N = 100000
_C = 128
_NBINS = 20

_BR = 2000
_NB = _N // _BR

_NC = 2
_NS = 16
_NW = _NC * _NS
_CHUNK = 3136
_NPAD = _CHUNK * _NW
_VECS = _CHUNK // 16
_UNROLL = 4


def _make_scale_kernel():
    mesh = plsc.VectorSubcoreMesh(core_axis_name="c", subcore_axis_name="s")

    @functools.partial(
        pl.kernel,
        mesh=mesh,
        out_type=jax.ShapeDtypeStruct((_NPAD,), jnp.float32),
        scratch_types=[
            pltpu.VMEM((_CHUNK,), jnp.int32),
            pltpu.VMEM((_CHUNK,), jnp.float32),
            pltpu.VMEM((32,), jnp.float32),
        ],
    )
    def scale_k(bins_hbm, temp_hbm, out_hbm, idx_v, t_v, temp_v):
        wid = lax.axis_index("s") * _NC + lax.axis_index("c")
        base = wid * _CHUNK
        pltpu.sync_copy(temp_hbm, temp_v)
        pltpu.sync_copy(bins_hbm.at[pl.ds(base, _CHUNK)], idx_v)
        t0 = temp_v[pl.ds(0, 16)]
        t1 = temp_v[pl.ds(16, 16)]
        inv0 = 1.0 / (jnp.maximum(t0, 0.0) + 1e-8)
        inv1 = 1.0 / (jnp.maximum(t1, 0.0) + 1e-8)

        def body(i, carry):
            for u in range(_UNROLL):
                off = (i * _UNROLL + u) * 16
                idx = idx_v[pl.ds(off, 16)]
                g0 = inv0.at[jnp.minimum(idx, 15)].get(mode="promise_in_bounds")
                g1 = inv1.at[jnp.maximum(idx - 16, 0)].get(mode="promise_in_bounds")
                t_v[pl.ds(off, 16)] = jnp.where(idx < 16, g0, g1)
            return carry

        lax.fori_loop(0, _VECS // _UNROLL, body, 0)
        pltpu.sync_copy(t_v, out_hbm.at[pl.ds(base, _CHUNK)])

    return scale_k


_scale_kernel = _make_scale_kernel()


def _tc_scale_body(scale_ref, logits_ref, out_ref):
    s_col = scale_ref[...].reshape(_BR, 1)
    out_ref[...] = logits_ref[...] * s_col


def kernel(logits, temperature, bin_assignments):
    bins = bin_assignments.astype(jnp.int32)
    bins_p = jnp.pad(bins, (0, _NPAD - _N))
    temp_p = jnp.pad(temperature.astype(jnp.float32), (0, 32 - _NBINS))
    scale = _scale_kernel(bins_p, temp_p)
    scale3d = scale[:_N].reshape(_NB, 1, _BR)
    out = pl.pallas_call(
        _tc_scale_body,
        grid=(_NB,),
        in_specs=[
            pl.BlockSpec((1, 1, _BR), lambda i: (i, 0, 0)),
            pl.BlockSpec((_BR, _C), lambda i: (i, 0)),
        ],
        out_specs=pl.BlockSpec((_BR, _C), lambda i: (i, 0)),
        out_shape=jax.ShapeDtypeStruct((_N, _C), jnp.float32),
    )(scale3d, logits)
    return out


# speedup vs baseline: 9.4623x; 1.3052x over previous
import functools

import jax
import jax.numpy as jnp
from jax import lax
from jax.experimental import pallas as pl
from jax.experimental.pallas import tpu as pltpu
from jax.experimental.pallas import tpu_sc as plsc

_N = 100000
_C = 128
_NBINS = 20

_BR = 5000
_NB = _N // _BR

_NC = 2
_NS = 16
_NW = _NC * _NS
_CHUNK = 3136
_LAST_BASE = _N - _CHUNK
_VECS = _CHUNK // 16
_UNROLL = 4


def _make_scale_kernel():
    mesh = plsc.VectorSubcoreMesh(core_axis_name="c", subcore_axis_name="s")

    @functools.partial(
        pl.kernel,
        mesh=mesh,
        out_type=jax.ShapeDtypeStruct((_N,), jnp.float32),
        scratch_types=[
            pltpu.VMEM((_CHUNK,), jnp.int32),
            pltpu.VMEM((_CHUNK,), jnp.float32),
            pltpu.VMEM((32,), jnp.float32),
        ],
    )
    def scale_k(bins_hbm, temp_hbm, out_hbm, idx_v, t_v, temp_v):
        wid = lax.axis_index("s") * _NC + lax.axis_index("c")
        base = jnp.minimum(wid * _CHUNK, _LAST_BASE)
        pltpu.sync_copy(temp_hbm, temp_v)
        pltpu.sync_copy(bins_hbm.at[pl.ds(base, _CHUNK)], idx_v)
        t0 = temp_v[pl.ds(0, 16)]
        t1 = temp_v[pl.ds(16, 16)]
        inv0 = 1.0 / (jnp.maximum(t0, 0.0) + 1e-8)
        inv1 = 1.0 / (jnp.maximum(t1, 0.0) + 1e-8)

        def body(i, carry):
            for u in range(_UNROLL):
                off = (i * _UNROLL + u) * 16
                idx = idx_v[pl.ds(off, 16)]
                g0 = inv0.at[jnp.minimum(idx, 15)].get(mode="promise_in_bounds")
                g1 = inv1.at[jnp.maximum(idx - 16, 0)].get(mode="promise_in_bounds")
                t_v[pl.ds(off, 16)] = jnp.where(idx < 16, g0, g1)
            return carry

        lax.fori_loop(0, _VECS // _UNROLL, body, 0)
        pltpu.sync_copy(t_v, out_hbm.at[pl.ds(base, _CHUNK)])

    return scale_k


_scale_kernel = _make_scale_kernel()


def _tc_scale_body(scale_ref, logits_ref, out_ref):
    s_col = scale_ref[...].reshape(_BR, 1)
    out_ref[...] = logits_ref[...] * s_col


def kernel(logits, temperature, bin_assignments):
    bins = bin_assignments.astype(jnp.int32)
    temp_p = jnp.pad(temperature.astype(jnp.float32), (0, 32 - _NBINS))
    scale = _scale_kernel(bins, temp_p)
    scale3d = scale.reshape(_NB, 1, _BR)
    out = pl.pallas_call(
        _tc_scale_body,
        grid=(_NB,),
        in_specs=[
            pl.BlockSpec((1, 1, _BR), lambda i: (i, 0, 0)),
            pl.BlockSpec((_BR, _C), lambda i: (i, 0)),
        ],
        out_specs=pl.BlockSpec((_BR, _C), lambda i: (i, 0)),
        out_shape=jax.ShapeDtypeStruct((_N, _C), jnp.float32),
    )(scale3d, logits)
    return out


# speedup vs baseline: 9.9402x; 1.0505x over previous
import functools

import jax
import jax.numpy as jnp
from jax import lax
from jax.experimental import pallas as pl
from jax.experimental.pallas import tpu as pltpu
from jax.experimental.pallas import tpu_sc as plsc

_N = 100000
_C = 128
_NBINS = 20

_BR = 10000
_NB = _N // _BR

_NC = 2
_NS = 16
_NW = _NC * _NS
_CHUNK = 3136
_LAST_BASE = _N - _CHUNK
_VECS = _CHUNK // 16
_UNROLL = 4


def _make_scale_kernel():
    mesh = plsc.VectorSubcoreMesh(core_axis_name="c", subcore_axis_name="s")

    @functools.partial(
        pl.kernel,
        mesh=mesh,
        out_type=jax.ShapeDtypeStruct((_N,), jnp.float32),
        scratch_types=[
            pltpu.VMEM((_CHUNK,), jnp.int32),
            pltpu.VMEM((_CHUNK,), jnp.float32),
            pltpu.VMEM((32,), jnp.float32),
        ],
    )
    def scale_k(bins_hbm, temp_hbm, out_hbm, idx_v, t_v, temp_v):
        wid = lax.axis_index("s") * _NC + lax.axis_index("c")
        base = jnp.minimum(wid * _CHUNK, _LAST_BASE)
        pltpu.sync_copy(temp_hbm, temp_v)
        pltpu.sync_copy(bins_hbm.at[pl.ds(base, _CHUNK)], idx_v)
        t0 = temp_v[pl.ds(0, 16)]
        t1 = temp_v[pl.ds(16, 16)]
        inv0 = 1.0 / (jnp.maximum(t0, 0.0) + 1e-8)
        inv1 = 1.0 / (jnp.maximum(t1, 0.0) + 1e-8)

        def body(i, carry):
            for u in range(_UNROLL):
                off = (i * _UNROLL + u) * 16
                idx = idx_v[pl.ds(off, 16)]
                g0 = inv0.at[jnp.minimum(idx, 15)].get(mode="promise_in_bounds")
                g1 = inv1.at[jnp.maximum(idx - 16, 0)].get(mode="promise_in_bounds")
                t_v[pl.ds(off, 16)] = jnp.where(idx < 16, g0, g1)
            return carry

        lax.fori_loop(0, _VECS // _UNROLL, body, 0)
        pltpu.sync_copy(t_v, out_hbm.at[pl.ds(base, _CHUNK)])

    return scale_k


_scale_kernel = _make_scale_kernel()


def _tc_scale_body(scale_ref, logits_ref, out_ref):
    s_col = scale_ref[...].reshape(_BR, 1)
    out_ref[...] = logits_ref[...] * s_col


def kernel(logits, temperature, bin_assignments):
    bins = bin_assignments.astype(jnp.int32)
    temp_p = jnp.pad(temperature.astype(jnp.float32), (0, 32 - _NBINS))
    scale = _scale_kernel(bins, temp_p)
    scale3d = scale.reshape(_NB, 1, _BR)
    out = pl.pallas_call(
        _tc_scale_body,
        grid=(_NB,),
        in_specs=[
            pl.BlockSpec((1, 1, _BR), lambda i: (i, 0, 0)),
            pl.BlockSpec((_BR, _C), lambda i: (i, 0)),
        ],
        out_specs=pl.BlockSpec((_BR, _C), lambda i: (i, 0)),
        out_shape=jax.ShapeDtypeStruct((_N, _C), jnp.float32),
    )(scale3d, logits)
    return out


# speedup vs baseline: 10.0585x; 1.0119x over previous
import functools

import jax
import jax.numpy as jnp
from jax import lax
from jax.experimental import pallas as pl
from jax.experimental.pallas import tpu as pltpu
from jax.experimental.pallas import tpu_sc as plsc

_N = 100000
_C = 128
_NBINS = 20

_BR = 20000
_NB = _N // _BR

_NC = 2
_NS = 16
_NW = _NC * _NS
_CHUNK = 3136
_LAST_BASE = _N - _CHUNK
_VECS = _CHUNK // 16
_UNROLL = 4


def _make_scale_kernel():
    mesh = plsc.VectorSubcoreMesh(core_axis_name="c", subcore_axis_name="s")

    @functools.partial(
        pl.kernel,
        mesh=mesh,
        out_type=jax.ShapeDtypeStruct((_N,), jnp.float32),
        scratch_types=[
            pltpu.VMEM((_CHUNK,), jnp.int32),
            pltpu.VMEM((_CHUNK,), jnp.float32),
            pltpu.VMEM((32,), jnp.float32),
        ],
    )
    def scale_k(bins_hbm, temp_hbm, out_hbm, idx_v, t_v, temp_v):
        wid = lax.axis_index("s") * _NC + lax.axis_index("c")
        base = jnp.minimum(wid * _CHUNK, _LAST_BASE)
        pltpu.sync_copy(temp_hbm, temp_v)
        pltpu.sync_copy(bins_hbm.at[pl.ds(base, _CHUNK)], idx_v)
        t0 = temp_v[pl.ds(0, 16)]
        t1 = temp_v[pl.ds(16, 16)]
        inv0 = 1.0 / (jnp.maximum(t0, 0.0) + 1e-8)
        inv1 = 1.0 / (jnp.maximum(t1, 0.0) + 1e-8)

        def body(i, carry):
            for u in range(_UNROLL):
                off = (i * _UNROLL + u) * 16
                idx = idx_v[pl.ds(off, 16)]
                g0 = inv0.at[jnp.minimum(idx, 15)].get(mode="promise_in_bounds")
                g1 = inv1.at[jnp.maximum(idx - 16, 0)].get(mode="promise_in_bounds")
                t_v[pl.ds(off, 16)] = jnp.where(idx < 16, g0, g1)
            return carry

        lax.fori_loop(0, _VECS // _UNROLL, body, 0)
        pltpu.sync_copy(t_v, out_hbm.at[pl.ds(base, _CHUNK)])

    return scale_k


_scale_kernel = _make_scale_kernel()


def _tc_scale_body(scale_ref, logits_ref, out_ref):
    s_col = scale_ref[...].reshape(_BR, 1)
    out_ref[...] = logits_ref[...] * s_col


def kernel(logits, temperature, bin_assignments):
    bins = bin_assignments.astype(jnp.int32)
    temp_p = jnp.pad(temperature.astype(jnp.float32), (0, 32 - _NBINS))
    scale = _scale_kernel(bins, temp_p)
    scale3d = scale.reshape(_NB, 1, _BR)
    out = pl.pallas_call(
        _tc_scale_body,
        grid=(_NB,),
        in_specs=[
            pl.BlockSpec((1, 1, _BR), lambda i: (i, 0, 0)),
            pl.BlockSpec((_BR, _C), lambda i: (i, 0)),
        ],
        out_specs=pl.BlockSpec((_BR, _C), lambda i: (i, 0)),
        out_shape=jax.ShapeDtypeStruct((_N, _C), jnp.float32),
    )(scale3d, logits)
    return out


# speedup vs baseline: 10.1367x; 1.0078x over previous
import functools

import jax
import jax.numpy as jnp
from jax import lax
from jax.experimental import pallas as pl
from jax.experimental.pallas import tpu as pltpu
from jax.experimental.pallas import tpu_sc as plsc

_N = 100000
_C = 128
_NBINS = 20

_BR = 25000
_NB = _N // _BR

_NC = 2
_NS = 16
_NW = _NC * _NS
_CHUNK = 3136
_LAST_BASE = _N - _CHUNK
_VECS = _CHUNK // 16
_UNROLL = 4


def _make_scale_kernel():
    mesh = plsc.VectorSubcoreMesh(core_axis_name="c", subcore_axis_name="s")

    @functools.partial(
        pl.kernel,
        mesh=mesh,
        out_type=jax.ShapeDtypeStruct((_N,), jnp.float32),
        scratch_types=[
            pltpu.VMEM((_CHUNK,), jnp.int32),
            pltpu.VMEM((_CHUNK,), jnp.float32),
            pltpu.VMEM((32,), jnp.float32),
        ],
    )
    def scale_k(bins_hbm, temp_hbm, out_hbm, idx_v, t_v, temp_v):
        wid = lax.axis_index("s") * _NC + lax.axis_index("c")
        base = jnp.minimum(wid * _CHUNK, _LAST_BASE)
        pltpu.sync_copy(temp_hbm, temp_v)
        pltpu.sync_copy(bins_hbm.at[pl.ds(base, _CHUNK)], idx_v)
        t0 = temp_v[pl.ds(0, 16)]
        t1 = temp_v[pl.ds(16, 16)]
        inv0 = 1.0 / (jnp.maximum(t0, 0.0) + 1e-8)
        inv1 = 1.0 / (jnp.maximum(t1, 0.0) + 1e-8)

        def body(i, carry):
            for u in range(_UNROLL):
                off = (i * _UNROLL + u) * 16
                idx = idx_v[pl.ds(off, 16)]
                g0 = inv0.at[jnp.minimum(idx, 15)].get(mode="promise_in_bounds")
                g1 = inv1.at[jnp.maximum(idx - 16, 0)].get(mode="promise_in_bounds")
                t_v[pl.ds(off, 16)] = jnp.where(idx < 16, g0, g1)
            return carry

        lax.fori_loop(0, _VECS // _UNROLL, body, 0)
        pltpu.sync_copy(t_v, out_hbm.at[pl.ds(base, _CHUNK)])

    return scale_k


_scale_kernel = _make_scale_kernel()


def _tc_scale_body(scale_ref, logits_ref, out_ref):
    s_col = scale_ref[...].reshape(_BR, 1)
    out_ref[...] = logits_ref[...] * s_col


def kernel(logits, temperature, bin_assignments):
    bins = bin_assignments.astype(jnp.int32)
    temp_p = jnp.pad(temperature.astype(jnp.float32), (0, 32 - _NBINS))
    scale = _scale_kernel(bins, temp_p)
    scale3d = scale.reshape(_NB, 1, _BR)
    out = pl.pallas_call(
        _tc_scale_body,
        grid=(_NB,),
        in_specs=[
            pl.BlockSpec((1, 1, _BR), lambda i: (i, 0, 0)),
            pl.BlockSpec((_BR, _C), lambda i: (i, 0)),
        ],
        out_specs=pl.BlockSpec((_BR, _C), lambda i: (i, 0)),
        out_shape=jax.ShapeDtypeStruct((_N, _C), jnp.float32),
    )(scale3d, logits)
    return out
